# Initial kernel scaffold; baseline (speedup 1.0000x reference)
import jax
import jax.numpy as jnp
from jax.experimental import pallas as pl


def kernel(x, W, gamma, beta, edge_index):
    raise NotImplementedError("write your pallas kernel here")


# speedup vs baseline: 3.8682x; 3.8682x over previous
import functools

import jax
import jax.numpy as jnp
from jax import lax
from jax.experimental import pallas as pl
from jax.experimental.pallas import tpu as pltpu
from jax.experimental.pallas import tpu_sc as plsc

N = 10000
E = 324000
K = 27
EK = E // K
INC = 128
OUTC = 128

NC = 2
NS = 16
NW = NC * NS

C = 128
T_PER_TILE = 10240
NCHUNK = T_PER_TILE // C
E_PAD = NW * T_PER_TILE

ROWS_PER_TILE = 632
ACC_R = NS * ROWS_PER_TILE

NB = 5
BLK = N // NB


def _matmul_body(x_ref, w_ref, y_ref):
    y_ref[...] = jnp.dot(
        x_ref[...], w_ref[0], preferred_element_type=jnp.float32
    )[None]


def _tc_matmul(x, W):
    return pl.pallas_call(
        _matmul_body,
        grid=(NB, K),
        in_specs=[
            pl.BlockSpec((BLK, INC), lambda nb, k: (nb, 0)),
            pl.BlockSpec((1, INC, OUTC), lambda nb, k: (k, 0, 0)),
        ],
        out_specs=pl.BlockSpec((1, BLK, OUTC), lambda nb, k: (k, nb, 0)),
        out_shape=jax.ShapeDtypeStruct((K, N, OUTC), jnp.float32),
    )(x, W)


def _sc_accum(y2, gidx3, dst3, zeros):
    mesh = plsc.VectorSubcoreMesh(
        core_axis_name="c", subcore_axis_name="s", num_cores=NC, num_subcores=NS
    )

    @functools.partial(
        pl.kernel,
        out_type=jax.ShapeDtypeStruct((NC, ACC_R, OUTC), jnp.float32),
        mesh=mesh,
        scratch_types=[
            pltpu.VMEM((C,), jnp.int32),
            pltpu.VMEM((C,), jnp.int32),
            pltpu.VMEM((C, OUTC), jnp.float32),
            pltpu.VMEM_SHARED((ACC_R, OUTC), jnp.float32),
            pltpu.SemaphoreType.DMA,
        ],
    )
    def sc_kernel(y_hbm, gidx_hbm, dst_hbm, zeros_hbm, out_hbm,
                  idx_v, dst_v, rows_v, acc_sh, sem):
        cid = lax.axis_index("c")
        sid = lax.axis_index("s")
        w = cid * NS + sid

        pltpu.sync_copy(zeros_hbm, acc_sh.at[pl.ds(sid * ROWS_PER_TILE, ROWS_PER_TILE)])
        plsc.subcore_barrier()

        def chunk(j, carry):
            pltpu.sync_copy(gidx_hbm.at[w, j], idx_v)
            pltpu.sync_copy(dst_hbm.at[w, j], dst_v)
            pltpu.async_copy(y_hbm.at[idx_v], rows_v, sem).wait()
            pltpu.sync_copy(rows_v, acc_sh.at[dst_v], add=True)
            return carry

        lax.fori_loop(0, NCHUNK, chunk, 0)
        plsc.subcore_barrier()

        pltpu.sync_copy(acc_sh.at[pl.ds(sid * ROWS_PER_TILE, ROWS_PER_TILE)],
                        out_hbm.at[cid, pl.ds(sid * ROWS_PER_TILE, ROWS_PER_TILE)])

    return sc_kernel(y2, gidx3, dst3, zeros)


def _bn_body(p_ref, g_ref, b_ref, o_ref):
    s = p_ref[0, :N, :] + p_ref[1, :N, :]
    mean = jnp.mean(s, axis=0, keepdims=True)
    var = jnp.mean((s - mean) ** 2, axis=0, keepdims=True)
    o_ref[...] = jnp.maximum(
        (s - mean) * lax.rsqrt(var + 1e-5) * g_ref[...] + b_ref[...], 0.0
    )


def _tc_bn_relu(parts, gamma, beta):
    return pl.pallas_call(
        _bn_body,
        out_shape=jax.ShapeDtypeStruct((N, OUTC), jnp.float32),
    )(parts, gamma.reshape(1, OUTC), beta.reshape(1, OUTC))


def kernel(x, W, gamma, beta, edge_index):
    src = edge_index[0]
    dst = edge_index[1]

    k_of_e = jnp.arange(E, dtype=jnp.int32) // EK
    gidx = src + N * k_of_e

    pad = E_PAD - E
    gidx_p = jnp.concatenate([gidx, jnp.zeros((pad,), jnp.int32)])
    dst_p = jnp.concatenate([dst, jnp.full((pad,), N, jnp.int32)])
    gidx3 = gidx_p.reshape(NW, NCHUNK, C)
    dst3 = dst_p.reshape(NW, NCHUNK, C)
    zeros = jnp.zeros((ROWS_PER_TILE, OUTC), jnp.float32)

    y = _tc_matmul(x, W).reshape(K * N, OUTC)
    parts = _sc_accum(y, gidx3, dst3, zeros)
    return _tc_bn_relu(parts, gamma, beta)


# speedup vs baseline: 4.8196x; 1.2460x over previous
import functools

import jax
import jax.numpy as jnp
from jax import lax
from jax.experimental import pallas as pl
from jax.experimental.pallas import tpu as pltpu
from jax.experimental.pallas import tpu_sc as plsc

N = 10000
E = 324000
K = 27
EK = E // K
INC = 128
OUTC = 128

NC = 2
NS = 16
NW = NC * NS

C = 128
NCHUNK = 80
T_PER_TILE = NCHUNK * C
E_PAD = NW * T_PER_TILE

ROWS_PER_TILE = 632
ACC_R = NS * ROWS_PER_TILE

NB = 5
BLK = N // NB


def _matmul_body(x_ref, w_ref, y_ref):
    y_ref[...] = jnp.dot(
        x_ref[...], w_ref[0], preferred_element_type=jnp.float32
    )[None]


def _tc_matmul(x, W):
    return pl.pallas_call(
        _matmul_body,
        grid=(NB, K),
        in_specs=[
            pl.BlockSpec((BLK, INC), lambda nb, k: (nb, 0)),
            pl.BlockSpec((1, INC, OUTC), lambda nb, k: (k, 0, 0)),
        ],
        out_specs=pl.BlockSpec((1, BLK, OUTC), lambda nb, k: (k, nb, 0)),
        out_shape=jax.ShapeDtypeStruct((K, N, OUTC), jnp.float32),
    )(x, W)


NBUF = 2
NGRP = NCHUNK // NBUF


def _sc_accum(y2, eidx4, zeros):
    mesh = plsc.VectorSubcoreMesh(
        core_axis_name="c", subcore_axis_name="s", num_cores=NC, num_subcores=NS
    )

    @functools.partial(
        pl.kernel,
        out_type=jax.ShapeDtypeStruct((NC, ACC_R, OUTC), jnp.float32),
        mesh=mesh,
        scratch_types=[
            pltpu.VMEM((NBUF, 2, C), jnp.int32),
            pltpu.VMEM((NBUF, C, OUTC), jnp.float32),
            pltpu.VMEM_SHARED((ACC_R, OUTC), jnp.float32),
            [pltpu.SemaphoreType.DMA] * NBUF,
            [pltpu.SemaphoreType.DMA] * NBUF,
            [pltpu.SemaphoreType.DMA] * NBUF,
        ],
    )
    def sc_kernel(y_hbm, eidx_hbm, zeros_hbm, out_hbm,
                  eidx_v, rows_v, acc_sh, semi, semg, sems):
        cid = lax.axis_index("c")
        sid = lax.axis_index("s")
        w = cid * NS + sid

        pltpu.sync_copy(zeros_hbm, acc_sh.at[pl.ds(sid * ROWS_PER_TILE, ROWS_PER_TILE)])
        plsc.subcore_barrier()

        def fire_idx(j, b):
            pltpu.async_copy(eidx_hbm.at[w, j], eidx_v.at[b], semi[b])

        def wait_idx(j, b):
            pltpu.make_async_copy(eidx_hbm.at[w, j], eidx_v.at[b], semi[b]).wait()

        def fire_gather(j, b):
            pltpu.async_copy(y_hbm.at[eidx_v.at[b, 0]], rows_v.at[b], semg[b])

        def wait_gather(j, b):
            pltpu.make_async_copy(y_hbm.at[eidx_v.at[b, 0]], rows_v.at[b], semg[b]).wait()

        def scatter(j, b):
            pltpu.async_copy(rows_v.at[b], acc_sh.at[eidx_v.at[b, 1]], sems[b], add=True)

        def wait_scatter(j, b):
            pltpu.make_async_copy(rows_v.at[b], acc_sh.at[eidx_v.at[b, 1]], sems[b]).wait()

        fire_idx(0, 0)
        fire_idx(1, 1)
        wait_idx(0, 0)
        fire_gather(0, 0)

        def step(j, b, fire_next_idx, fire_next_gather):
            bn = 1 - b
            if fire_next_gather:
                wait_idx(j + 1, bn)
                fire_gather(j + 1, bn)
            wait_gather(j, b)
            scatter(j, b)
            wait_scatter(j, b)
            if fire_next_idx:
                fire_idx(j + 2, b)

        def group(g, carry):
            j = g * 2
            step(j, 0, True, True)
            step(j + 1, 1, True, True)
            return carry

        lax.fori_loop(0, NGRP - 1, group, 0)
        j = (NGRP - 1) * 2
        step(j, 0, False, True)
        step(j + 1, 1, False, False)

        plsc.subcore_barrier()
        pltpu.sync_copy(acc_sh.at[pl.ds(sid * ROWS_PER_TILE, ROWS_PER_TILE)],
                        out_hbm.at[cid, pl.ds(sid * ROWS_PER_TILE, ROWS_PER_TILE)])

    return sc_kernel(y2, eidx4, zeros)


def _bn_body(p_ref, g_ref, b_ref, o_ref):
    s = p_ref[0, :N, :] + p_ref[1, :N, :]
    mean = jnp.mean(s, axis=0, keepdims=True)
    var = jnp.mean((s - mean) ** 2, axis=0, keepdims=True)
    o_ref[...] = jnp.maximum(
        (s - mean) * lax.rsqrt(var + 1e-5) * g_ref[...] + b_ref[...], 0.0
    )


def _tc_bn_relu(parts, gamma, beta):
    return pl.pallas_call(
        _bn_body,
        out_shape=jax.ShapeDtypeStruct((N, OUTC), jnp.float32),
    )(parts, gamma.reshape(1, OUTC), beta.reshape(1, OUTC))


def kernel(x, W, gamma, beta, edge_index):
    src = edge_index[0]
    dst = edge_index[1]

    k_of_e = jnp.arange(E, dtype=jnp.int32) // EK
    gidx = src + N * k_of_e

    pad = E_PAD - E
    gidx_p = jnp.concatenate([gidx, jnp.zeros((pad,), jnp.int32)])
    dst_p = jnp.concatenate([dst, jnp.full((pad,), N, jnp.int32)])
    eidx4 = jnp.stack(
        [gidx_p.reshape(NW, NCHUNK, C), dst_p.reshape(NW, NCHUNK, C)], axis=2
    )
    zeros = jnp.zeros((ROWS_PER_TILE, OUTC), jnp.float32)

    y = _tc_matmul(x, W).reshape(K * N, OUTC)
    parts = _sc_accum(y, eidx4, zeros)
    return _tc_bn_relu(parts, gamma, beta)


# speedup vs baseline: 5.2899x; 1.0976x over previous
import functools

import jax
import jax.numpy as jnp
from jax import lax
from jax.experimental import pallas as pl
from jax.experimental.pallas import tpu as pltpu
from jax.experimental.pallas import tpu_sc as plsc

N = 10000
E = 324000
K = 27
EK = E // K
INC = 128
OUTC = 128

NC = 2
NS = 16
NW = NC * NS

C = 128
TOTQ = 2560
E_PAD = TOTQ * C
NCH0 = 110
NCH1 = 50

ROWS_PER_TILE = 632
ACC_R = NS * ROWS_PER_TILE

NB = 5
BLK = N // NB


def _matmul_body(x_ref, w_ref, y_ref):
    y_ref[...] = jnp.dot(
        x_ref[...], w_ref[0], preferred_element_type=jnp.float32
    )[None]


def _tc_matmul(x, W):
    return pl.pallas_call(
        _matmul_body,
        grid=(NB, K),
        in_specs=[
            pl.BlockSpec((BLK, INC), lambda nb, k: (nb, 0)),
            pl.BlockSpec((1, INC, OUTC), lambda nb, k: (k, 0, 0)),
        ],
        out_specs=pl.BlockSpec((1, BLK, OUTC), lambda nb, k: (k, nb, 0)),
        out_shape=jax.ShapeDtypeStruct((K, N, OUTC), jnp.float32),
    )(x, W)


NBUF = 2


def _sc_accum(y2, eidx4, zeros):
    mesh = plsc.VectorSubcoreMesh(
        core_axis_name="c", subcore_axis_name="s", num_cores=NC, num_subcores=NS
    )

    @functools.partial(
        pl.kernel,
        out_type=jax.ShapeDtypeStruct((NC, ACC_R, OUTC), jnp.float32),
        mesh=mesh,
        scratch_types=[
            pltpu.VMEM((NBUF, 2, C), jnp.int32),
            pltpu.VMEM((NBUF, C, OUTC), jnp.float32),
            pltpu.VMEM_SHARED((ACC_R, OUTC), jnp.float32),
            [pltpu.SemaphoreType.DMA] * NBUF,
            [pltpu.SemaphoreType.DMA] * NBUF,
            [pltpu.SemaphoreType.DMA] * NBUF,
        ],
    )
    def sc_kernel(y_hbm, eidx_hbm, zeros_hbm, out_hbm,
                  eidx_v, rows_v, acc_sh, semi, semg, sems):
        cid = lax.axis_index("c")
        sid = lax.axis_index("s")
        base = jnp.where(cid == 0, sid * NCH0, NS * NCH0 + sid * NCH1)
        nch = jnp.where(cid == 0, NCH0, NCH1)
        ngrp = nch // NBUF

        pltpu.sync_copy(zeros_hbm, acc_sh.at[pl.ds(sid * ROWS_PER_TILE, ROWS_PER_TILE)])
        plsc.subcore_barrier()

        def fire_idx(j, b):
            pltpu.async_copy(eidx_hbm.at[base + j], eidx_v.at[b], semi[b])

        def wait_idx(j, b):
            pltpu.make_async_copy(eidx_hbm.at[base + j], eidx_v.at[b], semi[b]).wait()

        def fire_gather(j, b):
            pltpu.async_copy(y_hbm.at[eidx_v.at[b, 0]], rows_v.at[b], semg[b])

        def wait_gather(j, b):
            pltpu.make_async_copy(y_hbm.at[eidx_v.at[b, 0]], rows_v.at[b], semg[b]).wait()

        def scatter(j, b):
            pltpu.async_copy(rows_v.at[b], acc_sh.at[eidx_v.at[b, 1]], sems[b], add=True)

        def wait_scatter(j, b):
            pltpu.make_async_copy(rows_v.at[b], acc_sh.at[eidx_v.at[b, 1]], sems[b]).wait()

        fire_idx(0, 0)
        fire_idx(1, 1)
        wait_idx(0, 0)
        fire_gather(0, 0)

        def step(j, b, fire_next_idx, fire_next_gather):
            bn = 1 - b
            if fire_next_gather:
                wait_idx(j + 1, bn)
                fire_gather(j + 1, bn)
            wait_gather(j, b)
            scatter(j, b)
            wait_scatter(j, b)
            if fire_next_idx:
                fire_idx(j + 2, b)

        def group(g, carry):
            j = g * 2
            step(j, 0, True, True)
            step(j + 1, 1, True, True)
            return carry

        lax.fori_loop(0, ngrp - 1, group, 0)
        j = (ngrp - 1) * 2
        step(j, 0, False, True)
        step(j + 1, 1, False, False)

        plsc.subcore_barrier()
        pltpu.sync_copy(acc_sh.at[pl.ds(sid * ROWS_PER_TILE, ROWS_PER_TILE)],
                        out_hbm.at[cid, pl.ds(sid * ROWS_PER_TILE, ROWS_PER_TILE)])

    return sc_kernel(y2, eidx4, zeros)


def _bn_body(p_ref, g_ref, b_ref, o_ref):
    s = p_ref[0, :N, :] + p_ref[1, :N, :]
    mean = jnp.mean(s, axis=0, keepdims=True)
    var = jnp.mean((s - mean) ** 2, axis=0, keepdims=True)
    o_ref[...] = jnp.maximum(
        (s - mean) * lax.rsqrt(var + 1e-5) * g_ref[...] + b_ref[...], 0.0
    )


def _tc_bn_relu(parts, gamma, beta):
    return pl.pallas_call(
        _bn_body,
        out_shape=jax.ShapeDtypeStruct((N, OUTC), jnp.float32),
    )(parts, gamma.reshape(1, OUTC), beta.reshape(1, OUTC))


def kernel(x, W, gamma, beta, edge_index):
    src = edge_index[0]
    dst = edge_index[1]

    gidx = (src.reshape(K, EK)
            + (N * jnp.arange(K, dtype=jnp.int32))[:, None]).reshape(E)

    pad = E_PAD - E
    gidx_p = jnp.concatenate([gidx, jnp.zeros((pad,), jnp.int32)])
    dst_p = jnp.concatenate([dst, jnp.full((pad,), N, jnp.int32)])
    eidx4 = jnp.stack(
        [gidx_p.reshape(TOTQ, C), dst_p.reshape(TOTQ, C)], axis=1
    )
    zeros = jnp.zeros((ROWS_PER_TILE, OUTC), jnp.float32)

    y = _tc_matmul(x, W).reshape(K * N, OUTC)
    parts = _sc_accum(y, eidx4, zeros)
    return _tc_bn_relu(parts, gamma, beta)


# speedup vs baseline: 5.7246x; 1.0822x over previous
import functools

import jax
import jax.numpy as jnp
from jax import lax
from jax.experimental import pallas as pl
from jax.experimental.pallas import tpu as pltpu
from jax.experimental.pallas import tpu_sc as plsc

N = 10000
E = 324000
K = 27
EK = E // K
INC = 128
OUTC = 128

NC = 2
NS = 16
NW = NC * NS

C = 128
TOTQ = 2560
E_PAD = TOTQ * C
NCH0 = 138
NCH1 = 22

ROWS_PER_TILE = 632
ACC_R = NS * ROWS_PER_TILE

NB = 5
BLK = N // NB


def _matmul_body(x_ref, w_ref, y_ref):
    y_ref[...] = jnp.dot(
        x_ref[...], w_ref[0], preferred_element_type=jnp.float32
    )[None]


def _tc_matmul(x, W):
    return pl.pallas_call(
        _matmul_body,
        grid=(NB, K),
        in_specs=[
            pl.BlockSpec((BLK, INC), lambda nb, k: (nb, 0)),
            pl.BlockSpec((1, INC, OUTC), lambda nb, k: (k, 0, 0)),
        ],
        out_specs=pl.BlockSpec((1, BLK, OUTC), lambda nb, k: (k, nb, 0)),
        out_shape=jax.ShapeDtypeStruct((K, N, OUTC), jnp.float32),
    )(x, W)


NBUF = 2


def _sc_accum(y2, eidx4, zeros):
    mesh = plsc.VectorSubcoreMesh(
        core_axis_name="c", subcore_axis_name="s", num_cores=NC, num_subcores=NS
    )

    @functools.partial(
        pl.kernel,
        out_type=jax.ShapeDtypeStruct((NC, ACC_R, OUTC), jnp.float32),
        mesh=mesh,
        scratch_types=[
            pltpu.VMEM((NBUF, 2, C), jnp.int32),
            pltpu.VMEM((NBUF, C, OUTC), jnp.float32),
            pltpu.VMEM_SHARED((ACC_R, OUTC), jnp.float32),
            [pltpu.SemaphoreType.DMA] * NBUF,
            [pltpu.SemaphoreType.DMA] * NBUF,
            [pltpu.SemaphoreType.DMA] * NBUF,
        ],
    )
    def sc_kernel(y_hbm, eidx_hbm, zeros_hbm, out_hbm,
                  eidx_v, rows_v, acc_sh, semi, semg, sems):
        cid = lax.axis_index("c")
        sid = lax.axis_index("s")
        base = jnp.where(cid == 0, sid * NCH0, NS * NCH0 + sid * NCH1)
        nch = jnp.where(cid == 0, NCH0, NCH1)
        ngrp = nch // NBUF

        pltpu.sync_copy(zeros_hbm, acc_sh.at[pl.ds(sid * ROWS_PER_TILE, ROWS_PER_TILE)])
        plsc.subcore_barrier()

        def fire_idx(j, b):
            pltpu.async_copy(eidx_hbm.at[base + j], eidx_v.at[b], semi[b])

        def wait_idx(j, b):
            pltpu.make_async_copy(eidx_hbm.at[base + j], eidx_v.at[b], semi[b]).wait()

        def fire_gather(j, b):
            pltpu.async_copy(y_hbm.at[eidx_v.at[b, 0]], rows_v.at[b], semg[b])

        def wait_gather(j, b):
            pltpu.make_async_copy(y_hbm.at[eidx_v.at[b, 0]], rows_v.at[b], semg[b]).wait()

        def scatter(j, b):
            pltpu.async_copy(rows_v.at[b], acc_sh.at[eidx_v.at[b, 1]], sems[b], add=True)

        def wait_scatter(j, b):
            pltpu.make_async_copy(rows_v.at[b], acc_sh.at[eidx_v.at[b, 1]], sems[b]).wait()

        fire_idx(0, 0)
        fire_idx(1, 1)
        wait_idx(0, 0)
        fire_gather(0, 0)

        def step(j, b, fire_next_idx, fire_next_gather):
            bn = 1 - b
            if fire_next_gather:
                wait_idx(j + 1, bn)
                fire_gather(j + 1, bn)
            wait_gather(j, b)
            scatter(j, b)
            wait_scatter(j, b)
            if fire_next_idx:
                fire_idx(j + 2, b)

        def group(g, carry):
            j = g * 2
            step(j, 0, True, True)
            step(j + 1, 1, True, True)
            return carry

        lax.fori_loop(0, ngrp - 1, group, 0)
        j = (ngrp - 1) * 2
        step(j, 0, False, True)
        step(j + 1, 1, False, False)

        plsc.subcore_barrier()
        pltpu.sync_copy(acc_sh.at[pl.ds(sid * ROWS_PER_TILE, ROWS_PER_TILE)],
                        out_hbm.at[cid, pl.ds(sid * ROWS_PER_TILE, ROWS_PER_TILE)])

    return sc_kernel(y2, eidx4, zeros)


def _bn_body(p_ref, g_ref, b_ref, o_ref):
    s = p_ref[0, :N, :] + p_ref[1, :N, :]
    mean = jnp.mean(s, axis=0, keepdims=True)
    var = jnp.mean((s - mean) ** 2, axis=0, keepdims=True)
    o_ref[...] = jnp.maximum(
        (s - mean) * lax.rsqrt(var + 1e-5) * g_ref[...] + b_ref[...], 0.0
    )


def _tc_bn_relu(parts, gamma, beta):
    return pl.pallas_call(
        _bn_body,
        out_shape=jax.ShapeDtypeStruct((N, OUTC), jnp.float32),
    )(parts, gamma.reshape(1, OUTC), beta.reshape(1, OUTC))


def kernel(x, W, gamma, beta, edge_index):
    src = edge_index[0]
    dst = edge_index[1]

    gidx = (src.reshape(K, EK)
            + (N * jnp.arange(K, dtype=jnp.int32))[:, None]).reshape(E)

    pad = E_PAD - E
    gidx_p = jnp.concatenate([gidx, jnp.zeros((pad,), jnp.int32)])
    dst_p = jnp.concatenate([dst, jnp.full((pad,), N, jnp.int32)])
    eidx4 = jnp.stack(
        [gidx_p.reshape(TOTQ, C), dst_p.reshape(TOTQ, C)], axis=1
    )
    zeros = jnp.zeros((ROWS_PER_TILE, OUTC), jnp.float32)

    y = _tc_matmul(x, W).reshape(K * N, OUTC)
    parts = _sc_accum(y, eidx4, zeros)
    return _tc_bn_relu(parts, gamma, beta)
